# baseline (speedup 1.0000x reference)
import jax
import jax.numpy as jnp
from jax.experimental import pallas as pl

_EULER_GAMMA = 0.57721566490153286060
_EPS = 1e-13
_TWO_GAMMA = 2.0 * _EULER_GAMMA
_BB = 64


def _duck_body(e_ref, erc_ref, nrc_ref, r_ref, out_ref):
    i = pl.program_id(0)
    nsteps = pl.num_programs(0)
    d = e_ref.shape[1] // 2

    e = e_ref[...]
    el = e[:, None, :d]
    er = e[:, None, d:]
    r = r_ref[...]
    rl = r[:, :, :d]
    rr = r[:, :, d:]

    e1 = jnp.exp(-jnp.abs(er - rr))
    e2 = jnp.exp(-jnp.abs(el - rl))
    m = jnp.minimum(er, rr) - jnp.maximum(el, rl) - _TWO_GAMMA
    w = jnp.exp(m) / ((1.0 + e1) * (1.0 + e2))
    sp_i = jnp.log1p(w)

    x = (rr - rl) - _TWO_GAMMA
    sp_r = jnp.maximum(x, 0.0) + jnp.log1p(jnp.exp(-jnp.abs(x)))

    term = jnp.log((sp_r + _EPS) / (sp_i + _EPS))
    per_bk = jnp.sum(term, axis=-1)

    mask = (nrc_ref[...] >= 1) & (erc_ref[...] >= 1)
    s = jnp.sum(jnp.where(mask, per_bk, 0.0)).reshape(1, 1)

    @pl.when(i == 0)
    def _init():
        out_ref[...] = jnp.zeros_like(out_ref)

    out_ref[...] += s

    @pl.when(i == nsteps - 1)
    def _finish():
        out_ref[...] *= 1.0 / (nsteps * e_ref.shape[0] * nrc_ref.shape[1])


def kernel(entity_box, rel_box, entity_rel_counts, neighbor_rel_counts):
    b, k, _, d = rel_box.shape
    e2d = entity_box.reshape(b, 2 * d)
    r3d = rel_box.reshape(b, k, 2 * d)
    erc = entity_rel_counts.reshape(b, 1)
    out = pl.pallas_call(
        _duck_body,
        grid=(b // _BB,),
        in_specs=[
            pl.BlockSpec((_BB, 2 * d), lambda i: (i, 0)),
            pl.BlockSpec((_BB, 1), lambda i: (i, 0)),
            pl.BlockSpec((_BB, k), lambda i: (i, 0)),
            pl.BlockSpec((_BB, k, 2 * d), lambda i: (i, 0, 0)),
        ],
        out_specs=pl.BlockSpec((1, 1), lambda i: (0, 0)),
        out_shape=jax.ShapeDtypeStruct((1, 1), jnp.float32),
    )(e2d, erc, neighbor_rel_counts, r3d)
    return out.reshape(())


# speedup vs baseline: 3.3511x; 3.3511x over previous
import jax
import jax.numpy as jnp
from jax.experimental import pallas as pl
from jax.experimental.pallas import tpu as pltpu

_EULER_GAMMA = 0.57721566490153286060
_EPS = 1e-13
_LOG2E = 1.4426950408889634
_LN2 = 0.6931471805599453
_C2 = 2.0 * _EULER_GAMMA * _LOG2E
_EPS2 = _EPS / _LN2
_BB = 64


def _duck_body(e_ref, erc_ref, nrc_ref, r_hbm, out_ref, rlbuf, rrbuf, acc_ref, sem):
    i = pl.program_id(0)
    nsteps = pl.num_programs(0)
    bb, k, _, d = r_hbm.shape[0] // pl.num_programs(0), r_hbm.shape[1], 2, r_hbm.shape[3]
    slot = jax.lax.rem(i, 2)
    nxt = jax.lax.rem(i + 1, 2)

    def start_copies(block, s):
        pltpu.make_async_copy(
            r_hbm.at[pl.ds(block * bb, bb), :, 0, :], rlbuf.at[s], sem.at[s, 0]
        ).start()
        pltpu.make_async_copy(
            r_hbm.at[pl.ds(block * bb, bb), :, 1, :], rrbuf.at[s], sem.at[s, 1]
        ).start()

    @pl.when(i == 0)
    def _first():
        start_copies(0, 0)

    @pl.when(i + 1 < nsteps)
    def _prefetch():
        start_copies(i + 1, nxt)

    pltpu.make_async_copy(
        r_hbm.at[pl.ds(i * bb, bb), :, 0, :], rlbuf.at[slot], sem.at[slot, 0]
    ).wait()
    pltpu.make_async_copy(
        r_hbm.at[pl.ds(i * bb, bb), :, 1, :], rrbuf.at[slot], sem.at[slot, 1]
    ).wait()

    el = (e_ref[:, 0, :] * _LOG2E)[:, None, :]
    er = (e_ref[:, 1, :] * _LOG2E)[:, None, :]
    rl = rlbuf[slot] * _LOG2E
    rr = rrbuf[slot] * _LOG2E

    e1 = jnp.exp2(-jnp.abs(er - rr))
    e2 = jnp.exp2(-jnp.abs(el - rl))
    m2 = jnp.minimum(er, rr) - jnp.maximum(el, rl) - _C2
    w = jnp.exp2(m2) / ((1.0 + e1) * (1.0 + e2))
    p_i = jnp.log2(1.0 + w)

    x2 = (rr - rl) - _C2
    p_r = jnp.log2(1.0 + jnp.exp2(x2))

    term = jnp.log2(p_r + _EPS2) - jnp.log2(p_i + _EPS2)

    maskf = ((nrc_ref[...] >= 1) & (erc_ref[...] >= 1)).astype(jnp.float32)
    part = jnp.sum(term * maskf[:, :, None], axis=(0, 1))

    @pl.when(i == 0)
    def _init():
        acc_ref[...] = jnp.zeros_like(acc_ref)

    acc_ref[...] += part[None, :]

    @pl.when(i == nsteps - 1)
    def _finish():
        scale = _LN2 / (nsteps * bb * k)
        out_ref[...] = (jnp.sum(acc_ref[...]) * scale).reshape(1, 1)


def kernel(entity_box, rel_box, entity_rel_counts, neighbor_rel_counts):
    b, k, _, d = rel_box.shape
    erc = entity_rel_counts.reshape(b, 1)
    out = pl.pallas_call(
        _duck_body,
        grid=(b // _BB,),
        in_specs=[
            pl.BlockSpec((_BB, 2, d), lambda i: (i, 0, 0)),
            pl.BlockSpec((_BB, 1), lambda i: (i, 0)),
            pl.BlockSpec((_BB, k), lambda i: (i, 0)),
            pl.BlockSpec(memory_space=pltpu.MemorySpace.HBM),
        ],
        out_specs=pl.BlockSpec((1, 1), lambda i: (0, 0)),
        out_shape=jax.ShapeDtypeStruct((1, 1), jnp.float32),
        scratch_shapes=[
            pltpu.VMEM((2, _BB, k, d), jnp.float32),
            pltpu.VMEM((2, _BB, k, d), jnp.float32),
            pltpu.VMEM((1, d), jnp.float32),
            pltpu.SemaphoreType.DMA((2, 2)),
        ],
    )(entity_box, erc, neighbor_rel_counts, rel_box)
    return out.reshape(())


# speedup vs baseline: 3.5618x; 1.0629x over previous
import jax
import jax.numpy as jnp
from jax.experimental import pallas as pl
from jax.experimental.pallas import tpu as pltpu

_EULER_GAMMA = 0.57721566490153286060
_EPS = 1e-13
_LOG2E = 1.4426950408889634
_C2 = 2.0 * _EULER_GAMMA * _LOG2E
_BB = 64


def _duck_body(e_ref, erc_ref, nrc_ref, r_hbm, out_ref, rbuf, acc_ref, sem):
    i = pl.program_id(0)
    nsteps = pl.num_programs(0)
    b, k, _, d = r_hbm.shape
    bb = b // nsteps
    r_flat = r_hbm.reshape(b, k, 2 * d)
    slot = jax.lax.rem(i, 2)
    nxt = jax.lax.rem(i + 1, 2)

    def copy_for(block, s):
        return pltpu.make_async_copy(
            r_flat.at[pl.ds(block * bb, bb)], rbuf.at[s], sem.at[s]
        )

    @pl.when(i == 0)
    def _first():
        copy_for(0, 0).start()

    @pl.when(i + 1 < nsteps)
    def _prefetch():
        copy_for(i + 1, nxt).start()

    copy_for(i, slot).wait()

    el = (e_ref[:, 0, :] * _LOG2E)[:, None, :]
    er = (e_ref[:, 1, :] * _LOG2E)[:, None, :]
    r = rbuf[slot]
    rl = r[:, :, :d] * _LOG2E
    rr = r[:, :, d:] * _LOG2E

    e1 = jnp.exp2(-jnp.abs(er - rr))
    e2 = jnp.exp2(-jnp.abs(el - rl))
    m2 = jnp.minimum(er, rr) - jnp.maximum(el, rl) - _C2
    den = (1.0 + e1) * (1.0 + e2)
    sp_i = jnp.log(den + jnp.exp2(m2)) - jnp.log(den)

    x2 = (rr - rl) - _C2
    sp_r = jnp.log(1.0 + jnp.exp2(x2))

    term = jnp.log(sp_r + _EPS) - jnp.log(sp_i + _EPS)

    maskf = ((nrc_ref[...] >= 1) & (erc_ref[...] >= 1)).astype(jnp.float32)
    part = jnp.sum(term * maskf[:, :, None], axis=(0, 1))

    @pl.when(i == 0)
    def _init():
        acc_ref[...] = jnp.zeros_like(acc_ref)

    acc_ref[...] += part[None, :]

    @pl.when(i == nsteps - 1)
    def _finish():
        out_ref[...] = (jnp.sum(acc_ref[...]) / (nsteps * bb * k)).reshape(1, 1)


def kernel(entity_box, rel_box, entity_rel_counts, neighbor_rel_counts):
    b, k, _, d = rel_box.shape
    erc = entity_rel_counts.reshape(b, 1)
    out = pl.pallas_call(
        _duck_body,
        grid=(b // _BB,),
        in_specs=[
            pl.BlockSpec((_BB, 2, d), lambda i: (i, 0, 0)),
            pl.BlockSpec((_BB, 1), lambda i: (i, 0)),
            pl.BlockSpec((_BB, k), lambda i: (i, 0)),
            pl.BlockSpec(memory_space=pltpu.MemorySpace.HBM),
        ],
        out_specs=pl.BlockSpec((1, 1), lambda i: (0, 0)),
        out_shape=jax.ShapeDtypeStruct((1, 1), jnp.float32),
        scratch_shapes=[
            pltpu.VMEM((2, _BB, k, 2 * d), jnp.float32),
            pltpu.VMEM((1, d), jnp.float32),
            pltpu.SemaphoreType.DMA((2,)),
        ],
    )(entity_box, erc, neighbor_rel_counts, rel_box)
    return out.reshape(())


# speedup vs baseline: 3.6445x; 1.0232x over previous
import jax
import jax.numpy as jnp
from jax.experimental import pallas as pl
from jax.experimental.pallas import tpu as pltpu

_EULER_GAMMA = 0.57721566490153286060
_EPS = 1e-13
_LOG2E = 1.4426950408889634
_C2 = 2.0 * _EULER_GAMMA * _LOG2E
_BB = 128


def _duck_body(e_ref, erc_ref, nrc_ref, r_hbm, out_ref, rbuf, acc_ref, sem):
    i = pl.program_id(0)
    nsteps = pl.num_programs(0)
    b, k, _, d = r_hbm.shape
    bb = b // nsteps
    r_flat = r_hbm.reshape(b, k, 2 * d)
    slot = jax.lax.rem(i, 2)
    nxt = jax.lax.rem(i + 1, 2)

    def copy_for(block, s):
        return pltpu.make_async_copy(
            r_flat.at[pl.ds(block * bb, bb)], rbuf.at[s], sem.at[s]
        )

    @pl.when(i == 0)
    def _first():
        copy_for(0, 0).start()

    @pl.when(i + 1 < nsteps)
    def _prefetch():
        copy_for(i + 1, nxt).start()

    copy_for(i, slot).wait()

    el = (e_ref[:, 0, :] * _LOG2E + _C2)[:, None, :]
    er = (e_ref[:, 1, :] * _LOG2E)[:, None, :]
    r = rbuf[slot]
    rl = r[:, :, :d] * _LOG2E + _C2
    rr = r[:, :, d:] * _LOG2E

    e1 = jnp.exp2(-jnp.abs(er - rr))
    e2 = jnp.exp2(-jnp.abs(el - rl))
    m2 = jnp.minimum(er, rr) - jnp.maximum(el, rl)
    den = (1.0 + e1) * (1.0 + e2)
    sp_i = jnp.log(den + jnp.exp2(m2)) - jnp.log(den)

    x2 = rr - rl
    sp_r = jnp.log(1.0 + jnp.exp2(x2))

    maskf = ((nrc_ref[...] >= 1) & (erc_ref[...] >= 1)).astype(jnp.float32)
    mask3 = maskf[:, :, None] > 0.5
    pe_r = jnp.where(mask3, sp_r + _EPS, 1.0)
    pe_i = jnp.where(mask3, sp_i + _EPS, 1.0)
    pe_r = pe_r.reshape(pe_r.shape[0] // 2, 2, k, d)
    pe_i = pe_i.reshape(pe_i.shape[0] // 2, 2, k, d)
    pr = pe_r[:, 0] * pe_r[:, 1]
    pi = pe_i[:, 0] * pe_i[:, 1]
    part = jnp.sum(jnp.log(pr) - jnp.log(pi), axis=(0, 1))

    @pl.when(i == 0)
    def _init():
        acc_ref[...] = jnp.zeros_like(acc_ref)

    acc_ref[...] += part[None, :]

    @pl.when(i == nsteps - 1)
    def _finish():
        out_ref[...] = (jnp.sum(acc_ref[...]) / (nsteps * bb * k)).reshape(1, 1)


def kernel(entity_box, rel_box, entity_rel_counts, neighbor_rel_counts):
    b, k, _, d = rel_box.shape
    erc = entity_rel_counts.reshape(b, 1)
    out = pl.pallas_call(
        _duck_body,
        grid=(b // _BB,),
        in_specs=[
            pl.BlockSpec((_BB, 2, d), lambda i: (i, 0, 0)),
            pl.BlockSpec((_BB, 1), lambda i: (i, 0)),
            pl.BlockSpec((_BB, k), lambda i: (i, 0)),
            pl.BlockSpec(memory_space=pltpu.MemorySpace.HBM),
        ],
        out_specs=pl.BlockSpec((1, 1), lambda i: (0, 0)),
        out_shape=jax.ShapeDtypeStruct((1, 1), jnp.float32),
        scratch_shapes=[
            pltpu.VMEM((2, _BB, k, 2 * d), jnp.float32),
            pltpu.VMEM((1, d), jnp.float32),
            pltpu.SemaphoreType.DMA((2,)),
        ],
    )(entity_box, erc, neighbor_rel_counts, rel_box)
    return out.reshape(())
